# Initial kernel scaffold; baseline (speedup 1.0000x reference)
import jax
import jax.numpy as jnp
from jax.experimental import pallas as pl


def kernel(x, edge_index, edge_attr, W_emb, b_emb, W1, b1, W2, b2):
    raise NotImplementedError("write your pallas kernel here")


# speedup vs baseline: 1.0621x; 1.0621x over previous
import functools

import jax
import jax.numpy as jnp
from jax.experimental import pallas as pl

N_BLOCK = 2000


def _dense_body(s0_ref, s1_ref, dis_ref, u_ref, v_ref, b1_ref, w2_ref, b2_ref,
                hw2_ref, zd_ref):
    s0 = s0_ref[...]
    s1 = s1_ref[...]
    dis = dis_ref[...]
    u = u_ref[...]
    v = v_ref[...]
    b1 = b1_ref[...]
    h2 = jnp.maximum(s1 * u + s0 * v + b1, 0.0)
    hw2 = jnp.dot(h2, w2_ref[...], preferred_element_type=jnp.float32)
    hw2_ref[...] = hw2
    zd_ref[...] = dis * dis * hw2 + b2_ref[...]


def _dense_stage(s0, s1, dis, u, v, b1, W2, b2):
    n = s0.shape[0]
    grid = (n // N_BLOCK,)
    hw2, zd = pl.pallas_call(
        _dense_body,
        grid=grid,
        in_specs=[
            pl.BlockSpec((N_BLOCK, 1), lambda i: (i, 0)),
            pl.BlockSpec((N_BLOCK, 1), lambda i: (i, 0)),
            pl.BlockSpec((N_BLOCK, 1), lambda i: (i, 0)),
            pl.BlockSpec((1, 128), lambda i: (0, 0)),
            pl.BlockSpec((1, 128), lambda i: (0, 0)),
            pl.BlockSpec((1, 128), lambda i: (0, 0)),
            pl.BlockSpec((128, 64), lambda i: (0, 0)),
            pl.BlockSpec((1, 64), lambda i: (0, 0)),
        ],
        out_specs=[
            pl.BlockSpec((N_BLOCK, 64), lambda i: (i, 0)),
            pl.BlockSpec((N_BLOCK, 64), lambda i: (i, 0)),
        ],
        out_shape=[
            jax.ShapeDtypeStruct((n, 64), jnp.float32),
            jax.ShapeDtypeStruct((n, 64), jnp.float32),
        ],
    )(s0, s1, dis, u, v, b1, W2, b2)
    return hw2, zd


def kernel(x, edge_index, edge_attr, W_emb, b_emb, W1, b1, W2, b2):
    n = x.shape[0]
    row = edge_index[0]
    col = edge_index[1]
    ew = edge_attr
    xf = x[:, 0]

    deg = jnp.ones((n,), jnp.float32).at[col].add(ew)
    dis = jax.lax.rsqrt(deg)
    norm = dis[row] * ew * dis[col]

    s0 = jnp.zeros((n,), jnp.float32).at[col].add(norm) + dis * dis
    s1 = jnp.zeros((n,), jnp.float32).at[col].add(norm * xf[row]) + dis * dis * xf

    u = (W_emb[0] @ W1)[None, :]
    v = (b_emb @ W1)[None, :]

    hw2, zd = _dense_stage(s0[:, None], s1[:, None], dis[:, None],
                           u, v, b1[None, :], W2, b2[None, :])

    z = zd.at[col].add(norm[:, None] * hw2[row])
    return jnp.sum(z[row] * z[col], axis=1)


# speedup vs baseline: 15.9241x; 14.9932x over previous
import functools

import jax
import jax.numpy as jnp
from jax import lax
from jax.experimental import pallas as pl
from jax.experimental.pallas import tpu as pltpu
from jax.experimental.pallas import tpu_sc as plsc

N = 50000
E = 800000
NC = 2
NS = 16
NW = NC * NS

EPW = E // NW
EPT = E // NS
CH = 5000
CHP = CH + 16
CH5 = 400

CHD = 400
NPT = N // NS


def _vec_mesh():
    return plsc.VectorSubcoreMesh(core_axis_name="c", subcore_axis_name="s")


_SC_PARAMS = pltpu.CompilerParams(needs_layout_passes=False,
                                  use_tc_tiling_on_sc=False)


def _zero_f32(ref, n):
    def body(i, _):
        ref[pl.ds(i * 16, 16)] = jnp.zeros((16,), jnp.float32)
        return 0
    lax.fori_loop(0, n // 16, body, 0)


def _sc_deg_body(col_hbm, ew_hbm, degp_hbm, colb, ewb, acc, sem):
    cid = lax.axis_index("c")
    sid = lax.axis_index("s")
    wid = sid * NC + cid
    _zero_f32(acc, N)
    base = wid * EPW

    def chunk(ci, _):
        pltpu.async_copy(col_hbm.at[pl.ds(base + ci * CH, CH)], colb.at[pl.ds(0, CH)], sem).wait()
        pltpu.async_copy(ew_hbm.at[pl.ds(base + ci * CH, CH)], ewb.at[pl.ds(0, CH)], sem).wait()

        def body(i, _):
            cvec = colb[pl.ds(i * 16, 16)]
            wvec = ewb[pl.ds(i * 16, 16)]
            m = (i * 16 + lax.iota(jnp.int32, 16)) < CH
            plsc.addupdate_scatter(acc, [cvec], wvec, mask=m)
            return 0
        lax.fori_loop(0, (CH + 15) // 16, body, 0)
        return 0
    lax.fori_loop(0, EPW // CH, chunk, 0)
    pltpu.sync_copy(acc, degp_hbm.at[pl.ds(wid * N, N)])


def _sc_deg(col, ew):
    f = pl.kernel(
        _sc_deg_body,
        out_type=jax.ShapeDtypeStruct((NW * N,), jnp.float32),
        mesh=_vec_mesh(),
        compiler_params=_SC_PARAMS,
        scratch_types=[
            pltpu.VMEM((CHP,), jnp.int32),
            pltpu.VMEM((CHP,), jnp.float32),
            pltpu.VMEM((N,), jnp.float32),
            pltpu.SemaphoreType.DMA,
        ],
    )
    return f(col, ew)


def _sc_norm_body(row_hbm, col_hbm, ew_hbm, dis_hbm, norm_hbm, s0p_hbm,
                  rowb, colb, ewb, normb, disb, acc, sem):
    cid = lax.axis_index("c")
    sid = lax.axis_index("s")
    wid = sid * NC + cid
    pltpu.sync_copy(dis_hbm, disb)
    _zero_f32(acc, N)
    base = wid * EPW

    def chunk(ci, _):
        off = base + ci * CH
        pltpu.async_copy(row_hbm.at[pl.ds(off, CH)], rowb.at[pl.ds(0, CH)], sem).wait()
        pltpu.async_copy(col_hbm.at[pl.ds(off, CH)], colb.at[pl.ds(0, CH)], sem).wait()
        pltpu.async_copy(ew_hbm.at[pl.ds(off, CH)], ewb.at[pl.ds(0, CH)], sem).wait()

        def body(i, _):
            rvec = rowb[pl.ds(i * 16, 16)]
            cvec = colb[pl.ds(i * 16, 16)]
            wvec = ewb[pl.ds(i * 16, 16)]
            m = (i * 16 + lax.iota(jnp.int32, 16)) < CH
            dr = plsc.load_gather(disb, [rvec], mask=m)
            dc = plsc.load_gather(disb, [cvec], mask=m)
            nv = dr * wvec * dc
            normb[pl.ds(i * 16, 16)] = nv
            plsc.addupdate_scatter(acc, [cvec], nv, mask=m)
            return 0
        lax.fori_loop(0, (CH + 15) // 16, body, 0)
        pltpu.sync_copy(normb.at[pl.ds(0, CH)], norm_hbm.at[pl.ds(off, CH)])
        return 0
    lax.fori_loop(0, EPW // CH, chunk, 0)
    pltpu.sync_copy(acc, s0p_hbm.at[pl.ds(wid * N, N)])


def _sc_norm(row, col, ew, dis):
    f = pl.kernel(
        _sc_norm_body,
        out_type=[
            jax.ShapeDtypeStruct((E,), jnp.float32),
            jax.ShapeDtypeStruct((NW * N,), jnp.float32),
        ],
        mesh=_vec_mesh(),
        compiler_params=_SC_PARAMS,
        scratch_types=[
            pltpu.VMEM((CHP,), jnp.int32),
            pltpu.VMEM((CHP,), jnp.int32),
            pltpu.VMEM((CHP,), jnp.float32),
            pltpu.VMEM((CHP,), jnp.float32),
            pltpu.VMEM((N,), jnp.float32),
            pltpu.VMEM((N,), jnp.float32),
            pltpu.SemaphoreType.DMA,
        ],
    )
    return f(row, col, ew, dis)


def _sc_s1_body(row_hbm, col_hbm, norm_hbm, x_hbm, s1p_hbm,
                rowb, colb, normb, xb, acc, sem):
    cid = lax.axis_index("c")
    sid = lax.axis_index("s")
    wid = sid * NC + cid
    pltpu.sync_copy(x_hbm, xb)
    _zero_f32(acc, N)
    base = wid * EPW

    def chunk(ci, _):
        off = base + ci * CH
        pltpu.async_copy(row_hbm.at[pl.ds(off, CH)], rowb.at[pl.ds(0, CH)], sem).wait()
        pltpu.async_copy(col_hbm.at[pl.ds(off, CH)], colb.at[pl.ds(0, CH)], sem).wait()
        pltpu.async_copy(norm_hbm.at[pl.ds(off, CH)], normb.at[pl.ds(0, CH)], sem).wait()

        def body(i, _):
            rvec = rowb[pl.ds(i * 16, 16)]
            cvec = colb[pl.ds(i * 16, 16)]
            nvec = normb[pl.ds(i * 16, 16)]
            m = (i * 16 + lax.iota(jnp.int32, 16)) < CH
            xr = plsc.load_gather(xb, [rvec], mask=m)
            plsc.addupdate_scatter(acc, [cvec], nvec * xr, mask=m)
            return 0
        lax.fori_loop(0, (CH + 15) // 16, body, 0)
        return 0
    lax.fori_loop(0, EPW // CH, chunk, 0)
    pltpu.sync_copy(acc, s1p_hbm.at[pl.ds(wid * N, N)])


def _sc_s1(row, col, norm, x):
    f = pl.kernel(
        _sc_s1_body,
        out_type=jax.ShapeDtypeStruct((NW * N,), jnp.float32),
        mesh=_vec_mesh(),
        compiler_params=_SC_PARAMS,
        scratch_types=[
            pltpu.VMEM((CHP,), jnp.int32),
            pltpu.VMEM((CHP,), jnp.int32),
            pltpu.VMEM((CHP,), jnp.float32),
            pltpu.VMEM((N,), jnp.float32),
            pltpu.VMEM((N,), jnp.float32),
            pltpu.SemaphoreType.DMA,
        ],
    )
    return f(row, col, norm, x)


def _reduce32(parts, rsqrt_of_1p):
    def body(p_ref, o_ref):
        s = jnp.sum(p_ref[...], axis=0, keepdims=True)
        if rsqrt_of_1p:
            o_ref[...] = lax.rsqrt(1.0 + s)
        else:
            o_ref[...] = s
    return pl.pallas_call(
        body,
        grid=(1,),
        in_specs=[pl.BlockSpec((NW, N), lambda i: (0, 0))],
        out_specs=pl.BlockSpec((1, N), lambda i: (0, 0)),
        out_shape=jax.ShapeDtypeStruct((1, N), jnp.float32),
    )(parts)


def _dense_body(s0_ref, s1_ref, dis_ref, x_ref, u_ref, v_ref, b1_ref, w2_ref,
                b2_ref, *out_refs):
    d2 = dis_ref[...] * dis_ref[...]
    s0 = s0_ref[...] + d2
    s1 = s1_ref[...] + d2 * x_ref[...]
    h2 = jnp.maximum(s1 * u_ref[...] + s0 * v_ref[...] + b1_ref[...], 0.0)
    hw2 = jnp.dot(h2, w2_ref[...], preferred_element_type=jnp.float32)
    zd = d2 * hw2 + b2_ref[...]
    for q in range(4):
        out_refs[q][...] = hw2[:, q * 16:(q + 1) * 16]
        out_refs[4 + q][...] = zd[:, q * 16:(q + 1) * 16]


def _dense_stage(s0, s1, dis, x, u, v, b1, W2, b2):
    nb = 2000
    col_spec = pl.BlockSpec((nb, 1), lambda i: (i, 0))
    w_spec = lambda r, c: pl.BlockSpec((r, c), lambda i: (0, 0))
    quarter = pl.BlockSpec((nb, 16), lambda i: (i, 0))
    return pl.pallas_call(
        _dense_body,
        grid=(N // nb,),
        in_specs=[col_spec, col_spec, col_spec, col_spec,
                  w_spec(1, 128), w_spec(1, 128), w_spec(1, 128),
                  w_spec(128, 64), w_spec(1, 64)],
        out_specs=[quarter] * 8,
        out_shape=[jax.ShapeDtypeStruct((N, 16), jnp.float32)] * 8,
    )(s0, s1, dis, x, u, v, b1, W2, b2)


ZP = 400
NZP = N // ZP
NCH = EPT // CH5


def _sc_spmm_body(row_hbm, col_hbm, norm_hbm,
                  hw2q0, hw2q1, hw2q2, hw2q3, zdq0, zdq1, zdq2, zdq3,
                  zq0, zq1, zq2, zq3,
                  rowb0, colb0, normb0, rowb1, colb1, normb1,
                  rbuf0, rbuf1, zsh, semz, semi0, semi1, semg0, semg1):
    cid = lax.axis_index("c")
    sid = lax.axis_index("s")
    ebase = sid * EPT
    rowb = (rowb0, rowb1)
    colb = (colb0, colb1)
    normb = (normb0, normb1)
    rbuf = (rbuf0, rbuf1)
    semi = (semi0, semi1)
    semg = (semg0, semg1)

    def zinit(zd_hbm):
        for pi in range((NZP + NS - 1) // NS):
            p = sid + NS * pi

            @pl.when(p < NZP)
            def _():
                pltpu.async_copy(zd_hbm.at[pl.ds(p * ZP, ZP)],
                                 rbuf0.at[pl.ds(0, ZP)], semz).wait()
                pltpu.sync_copy(rbuf0.at[pl.ds(0, ZP)],
                                zsh.at[pl.ds(p * ZP, ZP)])

    def zout(z_hbm):
        for pi in range((NZP + NS - 1) // NS):
            p = sid + NS * pi

            @pl.when(p < NZP)
            def _():
                pltpu.sync_copy(zsh.at[pl.ds(p * ZP, ZP)],
                                rbuf0.at[pl.ds(0, ZP)])
                pltpu.async_copy(rbuf0.at[pl.ds(0, ZP)],
                                 z_hbm.at[pl.ds(p * ZP, ZP)], semz).wait()

    def scale_rows(buf, nrm):
        def grp(g, _):
            nv = nrm[pl.ds(g * 16, 16)]
            for j in range(16):
                e = g * 16 + j
                buf[e, pl.ds(0, 16)] = buf[e, pl.ds(0, 16)] * nv[j]
            return 0
        lax.fori_loop(0, CH5 // 16, grp, 0)

    def spmm(hw2_hbm):
        def issue_idx(j, b):
            off = ebase + j * CH5
            pltpu.async_copy(row_hbm.at[pl.ds(off, CH5)], rowb[b], semi[b])
            pltpu.async_copy(col_hbm.at[pl.ds(off, CH5)], colb[b], semi[b])
            pltpu.async_copy(norm_hbm.at[pl.ds(off, CH5)],
                             normb[b].at[pl.ds(0, CH5)], semi[b])

        def wait_idx(b):
            pltpu.make_async_copy(row_hbm.at[pl.ds(0, CH5)], rowb[b],
                                  semi[b]).wait()
            pltpu.make_async_copy(col_hbm.at[pl.ds(0, CH5)], colb[b],
                                  semi[b]).wait()
            pltpu.make_async_copy(norm_hbm.at[pl.ds(0, CH5)],
                                  normb[b].at[pl.ds(0, CH5)], semi[b]).wait()

        def issue_gather(b):
            pltpu.async_copy(hw2_hbm.at[rowb[b]], rbuf[b].at[pl.ds(0, CH5)],
                             semg[b])

        def wait_gather(b):
            pltpu.make_async_copy(hw2_hbm.at[pl.ds(0, CH5)],
                                  rbuf[b].at[pl.ds(0, CH5)], semg[b]).wait()

        def step(j, a):
            b = 1 - a
            wait_gather(a)

            @pl.when(j + 1 < NCH)
            def _():
                wait_idx(b)
                issue_gather(b)

            scale_rows(rbuf[a], normb[a])
            pltpu.sync_copy(rbuf[a].at[pl.ds(0, CH5)],
                            zsh.at[colb[a]], add=True)

            @pl.when(j + 2 < NCH)
            def _():
                issue_idx(j + 2, a)

        issue_idx(0, 0)
        issue_idx(1, 1)
        wait_idx(0)
        issue_gather(0)

        def pair(i, _):
            step(2 * i, 0)
            step(2 * i + 1, 1)
            return 0
        lax.fori_loop(0, NCH // 2, pair, 0)
        step(NCH - 1, (NCH - 1) % 2)

    def quarter_pass(zd_hbm, hw2_hbm, z_hbm):
        zinit(zd_hbm)
        plsc.subcore_barrier()
        spmm(hw2_hbm)
        plsc.subcore_barrier()
        zout(z_hbm)
        plsc.subcore_barrier()

    @pl.when(cid == 0)
    def _():
        quarter_pass(zdq0, hw2q0, zq0)
        quarter_pass(zdq1, hw2q1, zq1)

    @pl.when(cid == 1)
    def _():
        quarter_pass(zdq2, hw2q2, zq2)
        quarter_pass(zdq3, hw2q3, zq3)


def _sc_spmm(row, col, norm, hw2q, zdq):
    f = pl.kernel(
        _sc_spmm_body,
        out_type=[jax.ShapeDtypeStruct((N, 16), jnp.float32)] * 4,
        mesh=_vec_mesh(),
        compiler_params=_SC_PARAMS,
        scratch_types=[
            pltpu.VMEM((CH5,), jnp.int32),
            pltpu.VMEM((CH5,), jnp.int32),
            pltpu.VMEM((CH5 + 16,), jnp.float32),
            pltpu.VMEM((CH5,), jnp.int32),
            pltpu.VMEM((CH5,), jnp.int32),
            pltpu.VMEM((CH5 + 16,), jnp.float32),
            pltpu.VMEM((CH5 + 16, 16), jnp.float32),
            pltpu.VMEM((CH5 + 16, 16), jnp.float32),
            pltpu.VMEM_SHARED((N, 16), jnp.float32),
            pltpu.SemaphoreType.DMA,
            pltpu.SemaphoreType.DMA,
            pltpu.SemaphoreType.DMA,
            pltpu.SemaphoreType.DMA,
            pltpu.SemaphoreType.DMA,
        ],
    )
    return f(row, col, norm, *hw2q, *zdq)


NCHD = EPT // CHD


def _sc_dec_body(row_hbm, col_hbm, zq0, zq1, zq2, zq3, ppa_hbm, ppb_hbm,
                 rb0, cb0, rb1, cb1,
                 g0a0, g0b0, g0a1, g0b1, g1a0, g1b0, g1a1, g1b1,
                 pbuf0, pbuf1, semi0, semi1, semg0, semg1, semo):
    cid = lax.axis_index("c")
    sid = lax.axis_index("s")
    ebase = sid * EPT
    rb = (rb0, rb1)
    cb = (cb0, cb1)
    ga0 = (g0a0, g1a0)
    gb0 = (g0b0, g1b0)
    ga1 = (g0a1, g1a1)
    gb1 = (g0b1, g1b1)
    pbuf = (pbuf0, pbuf1)
    semi = (semi0, semi1)
    semg = (semg0, semg1)
    NG = CHD // 16

    def decode(zA_hbm, zB_hbm, pp_hbm):
        def issue_idx(j, b):
            off = ebase + j * CHD
            pltpu.async_copy(row_hbm.at[pl.ds(off, CHD)], rb[b], semi[b])
            pltpu.async_copy(col_hbm.at[pl.ds(off, CHD)], cb[b], semi[b])

        def wait_idx(b):
            pltpu.make_async_copy(row_hbm.at[pl.ds(0, CHD)], rb[b],
                                  semi[b]).wait()
            pltpu.make_async_copy(col_hbm.at[pl.ds(0, CHD)], cb[b],
                                  semi[b]).wait()

        def issue_gathers(b):
            pltpu.async_copy(zA_hbm.at[rb[b]], ga0[b], semg[b])
            pltpu.async_copy(zA_hbm.at[cb[b]], gb0[b], semg[b])
            pltpu.async_copy(zB_hbm.at[rb[b]], ga1[b], semg[b])
            pltpu.async_copy(zB_hbm.at[cb[b]], gb1[b], semg[b])

        def wait_gathers(b):
            for _ in range(4):
                pltpu.make_async_copy(zA_hbm.at[pl.ds(0, CHD)], ga0[b],
                                      semg[b]).wait()

        def step(j, a):
            b = 1 - a
            wait_gathers(a)

            @pl.when(j + 1 < NCHD)
            def _():
                wait_idx(b)
                issue_gathers(b)

            @pl.when(j >= 2)
            def _():
                pltpu.make_async_copy(row_hbm.at[pl.ds(0, CHD)],
                                      pbuf[a], semo).wait()

            pb = pbuf[a]
            xa0, xb0, xa1, xb1 = ga0[a], gb0[a], ga1[a], gb1[a]

            def grp(g, _):
                jvec = g * 16 + lax.iota(jnp.int32, 16)
                acc = jnp.zeros((16,), jnp.float32)
                for k in range(16):
                    kvec = jnp.full((16,), k, jnp.int32)
                    acc = acc + (plsc.load_gather(xa0, [jvec, kvec]) *
                                 plsc.load_gather(xb0, [jvec, kvec]))
                    acc = acc + (plsc.load_gather(xa1, [jvec, kvec]) *
                                 plsc.load_gather(xb1, [jvec, kvec]))
                pb[pl.ds(g * 16, 16)] = acc
                return 0
            lax.fori_loop(0, NG, grp, 0)

            off = ebase + j * CHD
            pltpu.async_copy(pb, pp_hbm.at[pl.ds(off, CHD)], semo)

            @pl.when(j + 2 < NCHD)
            def _():
                issue_idx(j + 2, a)

        issue_idx(0, 0)
        issue_idx(1, 1)
        wait_idx(0)
        issue_gathers(0)

        def pair(i, _):
            step(2 * i, 0)
            step(2 * i + 1, 1)
            return 0
        lax.fori_loop(0, NCHD // 2, pair, 0)
        step(NCHD - 1, (NCHD - 1) % 2)
        for _ in range(2):
            pltpu.make_async_copy(row_hbm.at[pl.ds(0, CHD)], pbuf0,
                                  semo).wait()

    @pl.when(cid == 0)
    def _():
        decode(zq0, zq1, ppa_hbm)

    @pl.when(cid == 1)
    def _():
        decode(zq2, zq3, ppb_hbm)


def _sc_decode(row, col, zq):
    f = pl.kernel(
        _sc_dec_body,
        out_type=[
            jax.ShapeDtypeStruct((E,), jnp.float32),
            jax.ShapeDtypeStruct((E,), jnp.float32),
        ],
        mesh=_vec_mesh(),
        compiler_params=_SC_PARAMS,
        scratch_types=[
            pltpu.VMEM((CHD,), jnp.int32),
            pltpu.VMEM((CHD,), jnp.int32),
            pltpu.VMEM((CHD,), jnp.int32),
            pltpu.VMEM((CHD,), jnp.int32),
            pltpu.VMEM((CHD, 16), jnp.float32),
            pltpu.VMEM((CHD, 16), jnp.float32),
            pltpu.VMEM((CHD, 16), jnp.float32),
            pltpu.VMEM((CHD, 16), jnp.float32),
            pltpu.VMEM((CHD, 16), jnp.float32),
            pltpu.VMEM((CHD, 16), jnp.float32),
            pltpu.VMEM((CHD, 16), jnp.float32),
            pltpu.VMEM((CHD, 16), jnp.float32),
            pltpu.VMEM((CHD,), jnp.float32),
            pltpu.VMEM((CHD,), jnp.float32),
            pltpu.SemaphoreType.DMA,
            pltpu.SemaphoreType.DMA,
            pltpu.SemaphoreType.DMA,
            pltpu.SemaphoreType.DMA,
            pltpu.SemaphoreType.DMA,
        ],
    )
    return f(row, col, *zq)

def _final_add(ppa, ppb):
    def body(a_ref, b_ref, o_ref):
        o_ref[...] = a_ref[...] + b_ref[...]
    eb = 80000
    spec = pl.BlockSpec((1, eb), lambda i: (0, i))
    return pl.pallas_call(
        body,
        grid=(E // eb,),
        in_specs=[spec, spec],
        out_specs=spec,
        out_shape=jax.ShapeDtypeStruct((1, E), jnp.float32),
    )(ppa, ppb)


def kernel(x, edge_index, edge_attr, W_emb, b_emb, W1, b1, W2, b2):
    row = edge_index[0]
    col = edge_index[1]
    ew = edge_attr
    xf = x[:, 0]

    degp = _sc_deg(col, ew).reshape(NW, N)
    dis_row = _reduce32(degp, True)
    dis_flat = dis_row.reshape(N)

    norm, s0p = _sc_norm(row, col, ew, dis_flat)
    s1p = _sc_s1(row, col, norm, xf)

    s0 = _reduce32(s0p.reshape(NW, N), False).reshape(N, 1)
    s1 = _reduce32(s1p.reshape(NW, N), False).reshape(N, 1)

    u = (W_emb[0] @ W1)[None, :]
    v = (b_emb @ W1)[None, :]
    outs = _dense_stage(s0, s1, dis_row.reshape(N, 1), x,
                        u, v, b1[None, :], W2, b2[None, :])
    hw2q, zdq = outs[:4], outs[4:]

    zq = _sc_spmm(row, col, norm, hw2q, zdq)
    ppa, ppb = _sc_decode(row, col, zq)
    return _final_add(ppa.reshape(1, E), ppb.reshape(1, E))[0]


# speedup vs baseline: 16.2085x; 1.0179x over previous
import functools

import jax
import jax.numpy as jnp
from jax import lax
from jax.experimental import pallas as pl
from jax.experimental.pallas import tpu as pltpu
from jax.experimental.pallas import tpu_sc as plsc

N = 50000
E = 800000
NC = 2
NS = 16
NW = NC * NS

EPW = E // NW
EPT = E // NS
CH = 5000
CHP = CH + 16
CH5 = 400

CHD = 400
NPT = N // NS


def _vec_mesh():
    return plsc.VectorSubcoreMesh(core_axis_name="c", subcore_axis_name="s")


_SC_PARAMS = pltpu.CompilerParams(needs_layout_passes=False,
                                  use_tc_tiling_on_sc=False)


def _zero_f32(ref, n):
    def body(i, _):
        ref[pl.ds(i * 16, 16)] = jnp.zeros((16,), jnp.float32)
        return 0
    lax.fori_loop(0, n // 16, body, 0)


def _sc_deg_body(col_hbm, ew_hbm, degp_hbm, colb, ewb, acc, sem):
    cid = lax.axis_index("c")
    sid = lax.axis_index("s")
    wid = sid * NC + cid
    _zero_f32(acc, N)
    base = wid * EPW

    def chunk(ci, _):
        pltpu.async_copy(col_hbm.at[pl.ds(base + ci * CH, CH)], colb.at[pl.ds(0, CH)], sem).wait()
        pltpu.async_copy(ew_hbm.at[pl.ds(base + ci * CH, CH)], ewb.at[pl.ds(0, CH)], sem).wait()

        def body(i, _):
            cvec = colb[pl.ds(i * 16, 16)]
            wvec = ewb[pl.ds(i * 16, 16)]
            m = (i * 16 + lax.iota(jnp.int32, 16)) < CH
            plsc.addupdate_scatter(acc, [cvec], wvec, mask=m)
            return 0
        lax.fori_loop(0, (CH + 15) // 16, body, 0)
        return 0
    lax.fori_loop(0, EPW // CH, chunk, 0)
    pltpu.sync_copy(acc, degp_hbm.at[pl.ds(wid * N, N)])


def _sc_deg(col, ew):
    f = pl.kernel(
        _sc_deg_body,
        out_type=jax.ShapeDtypeStruct((NW * N,), jnp.float32),
        mesh=_vec_mesh(),
        compiler_params=_SC_PARAMS,
        scratch_types=[
            pltpu.VMEM((CHP,), jnp.int32),
            pltpu.VMEM((CHP,), jnp.float32),
            pltpu.VMEM((N,), jnp.float32),
            pltpu.SemaphoreType.DMA,
        ],
    )
    return f(col, ew)


def _sc_norm_body(row_hbm, col_hbm, ew_hbm, dis_hbm, norm_hbm, s0p_hbm,
                  rowb, colb, ewb, normb, disb, acc, sem):
    cid = lax.axis_index("c")
    sid = lax.axis_index("s")
    wid = sid * NC + cid
    pltpu.sync_copy(dis_hbm, disb)
    _zero_f32(acc, N)
    base = wid * EPW

    def chunk(ci, _):
        off = base + ci * CH
        pltpu.async_copy(row_hbm.at[pl.ds(off, CH)], rowb.at[pl.ds(0, CH)], sem).wait()
        pltpu.async_copy(col_hbm.at[pl.ds(off, CH)], colb.at[pl.ds(0, CH)], sem).wait()
        pltpu.async_copy(ew_hbm.at[pl.ds(off, CH)], ewb.at[pl.ds(0, CH)], sem).wait()

        def body(i, _):
            rvec = rowb[pl.ds(i * 16, 16)]
            cvec = colb[pl.ds(i * 16, 16)]
            wvec = ewb[pl.ds(i * 16, 16)]
            m = (i * 16 + lax.iota(jnp.int32, 16)) < CH
            dr = plsc.load_gather(disb, [rvec], mask=m)
            dc = plsc.load_gather(disb, [cvec], mask=m)
            nv = dr * wvec * dc
            normb[pl.ds(i * 16, 16)] = nv
            plsc.addupdate_scatter(acc, [cvec], nv, mask=m)
            return 0
        lax.fori_loop(0, (CH + 15) // 16, body, 0)
        pltpu.sync_copy(normb.at[pl.ds(0, CH)], norm_hbm.at[pl.ds(off, CH)])
        return 0
    lax.fori_loop(0, EPW // CH, chunk, 0)
    pltpu.sync_copy(acc, s0p_hbm.at[pl.ds(wid * N, N)])


def _sc_norm(row, col, ew, dis):
    f = pl.kernel(
        _sc_norm_body,
        out_type=[
            jax.ShapeDtypeStruct((E,), jnp.float32),
            jax.ShapeDtypeStruct((NW * N,), jnp.float32),
        ],
        mesh=_vec_mesh(),
        compiler_params=_SC_PARAMS,
        scratch_types=[
            pltpu.VMEM((CHP,), jnp.int32),
            pltpu.VMEM((CHP,), jnp.int32),
            pltpu.VMEM((CHP,), jnp.float32),
            pltpu.VMEM((CHP,), jnp.float32),
            pltpu.VMEM((N,), jnp.float32),
            pltpu.VMEM((N,), jnp.float32),
            pltpu.SemaphoreType.DMA,
        ],
    )
    return f(row, col, ew, dis)


def _sc_s1_body(row_hbm, col_hbm, norm_hbm, x_hbm, s1p_hbm,
                rowb, colb, normb, xb, acc, sem):
    cid = lax.axis_index("c")
    sid = lax.axis_index("s")
    wid = sid * NC + cid
    pltpu.sync_copy(x_hbm, xb)
    _zero_f32(acc, N)
    base = wid * EPW

    def chunk(ci, _):
        off = base + ci * CH
        pltpu.async_copy(row_hbm.at[pl.ds(off, CH)], rowb.at[pl.ds(0, CH)], sem).wait()
        pltpu.async_copy(col_hbm.at[pl.ds(off, CH)], colb.at[pl.ds(0, CH)], sem).wait()
        pltpu.async_copy(norm_hbm.at[pl.ds(off, CH)], normb.at[pl.ds(0, CH)], sem).wait()

        def body(i, _):
            rvec = rowb[pl.ds(i * 16, 16)]
            cvec = colb[pl.ds(i * 16, 16)]
            nvec = normb[pl.ds(i * 16, 16)]
            m = (i * 16 + lax.iota(jnp.int32, 16)) < CH
            xr = plsc.load_gather(xb, [rvec], mask=m)
            plsc.addupdate_scatter(acc, [cvec], nvec * xr, mask=m)
            return 0
        lax.fori_loop(0, (CH + 15) // 16, body, 0)
        return 0
    lax.fori_loop(0, EPW // CH, chunk, 0)
    pltpu.sync_copy(acc, s1p_hbm.at[pl.ds(wid * N, N)])


def _sc_s1(row, col, norm, x):
    f = pl.kernel(
        _sc_s1_body,
        out_type=jax.ShapeDtypeStruct((NW * N,), jnp.float32),
        mesh=_vec_mesh(),
        compiler_params=_SC_PARAMS,
        scratch_types=[
            pltpu.VMEM((CHP,), jnp.int32),
            pltpu.VMEM((CHP,), jnp.int32),
            pltpu.VMEM((CHP,), jnp.float32),
            pltpu.VMEM((N,), jnp.float32),
            pltpu.VMEM((N,), jnp.float32),
            pltpu.SemaphoreType.DMA,
        ],
    )
    return f(row, col, norm, x)


def _reduce32(parts, rsqrt_of_1p):
    def body(p_ref, o_ref):
        s = jnp.sum(p_ref[...], axis=0, keepdims=True)
        if rsqrt_of_1p:
            o_ref[...] = lax.rsqrt(1.0 + s)
        else:
            o_ref[...] = s
    return pl.pallas_call(
        body,
        grid=(1,),
        in_specs=[pl.BlockSpec((NW, N), lambda i: (0, 0))],
        out_specs=pl.BlockSpec((1, N), lambda i: (0, 0)),
        out_shape=jax.ShapeDtypeStruct((1, N), jnp.float32),
    )(parts)


def _dense_body(s0_ref, s1_ref, dis_ref, x_ref, u_ref, v_ref, b1_ref, w2_ref,
                b2_ref, *out_refs):
    d2 = dis_ref[...] * dis_ref[...]
    s0 = s0_ref[...] + d2
    s1 = s1_ref[...] + d2 * x_ref[...]
    h2 = jnp.maximum(s1 * u_ref[...] + s0 * v_ref[...] + b1_ref[...], 0.0)
    hw2 = jnp.dot(h2, w2_ref[...], preferred_element_type=jnp.float32)
    zd = d2 * hw2 + b2_ref[...]
    for q in range(4):
        out_refs[q][...] = hw2[:, q * 16:(q + 1) * 16]
        out_refs[4 + q][...] = zd[:, q * 16:(q + 1) * 16]


def _dense_stage(s0, s1, dis, x, u, v, b1, W2, b2):
    nb = 2000
    col_spec = pl.BlockSpec((nb, 1), lambda i: (i, 0))
    w_spec = lambda r, c: pl.BlockSpec((r, c), lambda i: (0, 0))
    quarter = pl.BlockSpec((nb, 16), lambda i: (i, 0))
    return pl.pallas_call(
        _dense_body,
        grid=(N // nb,),
        in_specs=[col_spec, col_spec, col_spec, col_spec,
                  w_spec(1, 128), w_spec(1, 128), w_spec(1, 128),
                  w_spec(128, 64), w_spec(1, 64)],
        out_specs=[quarter] * 8,
        out_shape=[jax.ShapeDtypeStruct((N, 16), jnp.float32)] * 8,
    )(s0, s1, dis, x, u, v, b1, W2, b2)


ZP = 400
NZP = N // ZP
NCH = EPT // CH5
NCHD = EPT // CHD


def _sc_spmm_body(row_hbm, col_hbm, norm_hbm,
                  hw2q0, hw2q1, hw2q2, hw2q3, zdq0, zdq1, zdq2, zdq3,
                  pp0, pp1, pp2, pp3,
                  rowb0, colb0, normb0, rowb1, colb1, normb1,
                  rbuf0, rbuf1, pbuf0, pbuf1, zsh,
                  semz, semi0, semi1, semg0, semg1, semo):
    cid = lax.axis_index("c")
    sid = lax.axis_index("s")
    ebase = sid * EPT
    rowb = (rowb0, rowb1)
    colb = (colb0, colb1)
    normb = (normb0, normb1)
    rbuf = (rbuf0, rbuf1)
    pbuf = (pbuf0, pbuf1)
    semi = (semi0, semi1)
    semg = (semg0, semg1)

    def zinit(zd_hbm):
        for pi in range((NZP + NS - 1) // NS):
            p = sid + NS * pi

            @pl.when(p < NZP)
            def _():
                pltpu.async_copy(zd_hbm.at[pl.ds(p * ZP, ZP)],
                                 rbuf0.at[pl.ds(0, ZP)], semz).wait()
                pltpu.sync_copy(rbuf0.at[pl.ds(0, ZP)],
                                zsh.at[pl.ds(p * ZP, ZP)])

    def decode(pp_hbm):
        def d_issue_idx(j, b):
            off = ebase + j * CHD
            pltpu.async_copy(row_hbm.at[pl.ds(off, CHD)], rowb[b], semi[b])
            pltpu.async_copy(col_hbm.at[pl.ds(off, CHD)], colb[b], semi[b])

        def d_wait_idx(b):
            pltpu.make_async_copy(row_hbm.at[pl.ds(0, CHD)], rowb[b],
                                  semi[b]).wait()
            pltpu.make_async_copy(col_hbm.at[pl.ds(0, CHD)], colb[b],
                                  semi[b]).wait()

        def d_step(j, a):
            d_wait_idx(a)
            pltpu.async_copy(zsh.at[rowb[a]], rbuf0.at[pl.ds(0, CHD)], semg0)
            pltpu.async_copy(zsh.at[colb[a]], rbuf1.at[pl.ds(0, CHD)], semg1)
            pltpu.make_async_copy(zsh.at[pl.ds(0, CHD)],
                                  rbuf0.at[pl.ds(0, CHD)], semg0).wait()
            pltpu.make_async_copy(zsh.at[pl.ds(0, CHD)],
                                  rbuf1.at[pl.ds(0, CHD)], semg1).wait()

            @pl.when(j + 2 < NCHD)
            def _():
                d_issue_idx(j + 2, a)

            @pl.when(j >= 2)
            def _():
                pltpu.make_async_copy(row_hbm.at[pl.ds(0, CHD)],
                                      pbuf[a], semo).wait()

            pb = pbuf[a]

            def grp(g, _):
                jvec = g * 16 + lax.iota(jnp.int32, 16)
                acc = jnp.zeros((16,), jnp.float32)
                for k in range(16):
                    kvec = jnp.full((16,), k, jnp.int32)
                    acc = acc + (plsc.load_gather(rbuf0, [jvec, kvec]) *
                                 plsc.load_gather(rbuf1, [jvec, kvec]))
                pb[pl.ds(g * 16, 16)] = acc
                return 0
            lax.fori_loop(0, CHD // 16, grp, 0)

            pltpu.async_copy(pb, pp_hbm.at[pl.ds(ebase + j * CHD, CHD)], semo)

        d_issue_idx(0, 0)
        d_issue_idx(1, 1)

        def d_pair(i, _):
            d_step(2 * i, 0)
            d_step(2 * i + 1, 1)
            return 0
        lax.fori_loop(0, NCHD // 2, d_pair, 0)
        d_step(NCHD - 1, (NCHD - 1) % 2)
        for _ in range(2):
            pltpu.make_async_copy(row_hbm.at[pl.ds(0, CHD)], pbuf0,
                                  semo).wait()

    def scale_rows(buf, nrm):
        def grp(g, _):
            nv = nrm[pl.ds(g * 16, 16)]
            for j in range(16):
                e = g * 16 + j
                buf[e, pl.ds(0, 16)] = buf[e, pl.ds(0, 16)] * nv[j]
            return 0
        lax.fori_loop(0, CH5 // 16, grp, 0)

    def spmm(hw2_hbm):
        def issue_idx(j, b):
            off = ebase + j * CH5
            pltpu.async_copy(row_hbm.at[pl.ds(off, CH5)], rowb[b], semi[b])
            pltpu.async_copy(col_hbm.at[pl.ds(off, CH5)], colb[b], semi[b])
            pltpu.async_copy(norm_hbm.at[pl.ds(off, CH5)],
                             normb[b].at[pl.ds(0, CH5)], semi[b])

        def wait_idx(b):
            pltpu.make_async_copy(row_hbm.at[pl.ds(0, CH5)], rowb[b],
                                  semi[b]).wait()
            pltpu.make_async_copy(col_hbm.at[pl.ds(0, CH5)], colb[b],
                                  semi[b]).wait()
            pltpu.make_async_copy(norm_hbm.at[pl.ds(0, CH5)],
                                  normb[b].at[pl.ds(0, CH5)], semi[b]).wait()

        def issue_gather(b):
            pltpu.async_copy(hw2_hbm.at[rowb[b]], rbuf[b].at[pl.ds(0, CH5)],
                             semg[b])

        def wait_gather(b):
            pltpu.make_async_copy(hw2_hbm.at[pl.ds(0, CH5)],
                                  rbuf[b].at[pl.ds(0, CH5)], semg[b]).wait()

        def step(j, a):
            b = 1 - a
            wait_gather(a)

            @pl.when(j + 1 < NCH)
            def _():
                wait_idx(b)
                issue_gather(b)

            scale_rows(rbuf[a], normb[a])
            pltpu.sync_copy(rbuf[a].at[pl.ds(0, CH5)],
                            zsh.at[colb[a]], add=True)

            @pl.when(j + 2 < NCH)
            def _():
                issue_idx(j + 2, a)

        issue_idx(0, 0)
        issue_idx(1, 1)
        wait_idx(0)
        issue_gather(0)

        def pair(i, _):
            step(2 * i, 0)
            step(2 * i + 1, 1)
            return 0
        lax.fori_loop(0, NCH // 2, pair, 0)
        step(NCH - 1, (NCH - 1) % 2)

    def quarter_pass(zd_hbm, hw2_hbm, pp_hbm):
        zinit(zd_hbm)
        plsc.subcore_barrier()
        spmm(hw2_hbm)
        plsc.subcore_barrier()
        decode(pp_hbm)
        plsc.subcore_barrier()

    @pl.when(cid == 0)
    def _():
        quarter_pass(zdq0, hw2q0, pp0)
        quarter_pass(zdq1, hw2q1, pp1)

    @pl.when(cid == 1)
    def _():
        quarter_pass(zdq2, hw2q2, pp2)
        quarter_pass(zdq3, hw2q3, pp3)


def _sc_spmm(row, col, norm, hw2q, zdq):
    f = pl.kernel(
        _sc_spmm_body,
        out_type=[jax.ShapeDtypeStruct((E,), jnp.float32)] * 4,
        mesh=_vec_mesh(),
        compiler_params=_SC_PARAMS,
        scratch_types=[
            pltpu.VMEM((CH5,), jnp.int32),
            pltpu.VMEM((CH5,), jnp.int32),
            pltpu.VMEM((CH5 + 16,), jnp.float32),
            pltpu.VMEM((CH5,), jnp.int32),
            pltpu.VMEM((CH5,), jnp.int32),
            pltpu.VMEM((CH5 + 16,), jnp.float32),
            pltpu.VMEM((CH5 + 16, 16), jnp.float32),
            pltpu.VMEM((CH5 + 16, 16), jnp.float32),
            pltpu.VMEM((CHD,), jnp.float32),
            pltpu.VMEM((CHD,), jnp.float32),
            pltpu.VMEM_SHARED((N, 16), jnp.float32),
            pltpu.SemaphoreType.DMA,
            pltpu.SemaphoreType.DMA,
            pltpu.SemaphoreType.DMA,
            pltpu.SemaphoreType.DMA,
            pltpu.SemaphoreType.DMA,
            pltpu.SemaphoreType.DMA,
        ],
    )
    return f(row, col, norm, *hw2q, *zdq)


def _final_add(pp0, pp1, pp2, pp3):
    def body(a_ref, b_ref, c_ref, d_ref, o_ref):
        o_ref[...] = ((a_ref[...] + b_ref[...]) +
                      (c_ref[...] + d_ref[...]))
    eb = 80000
    spec = pl.BlockSpec((1, eb), lambda i: (0, i))
    return pl.pallas_call(
        body,
        grid=(E // eb,),
        in_specs=[spec] * 4,
        out_specs=spec,
        out_shape=jax.ShapeDtypeStruct((1, E), jnp.float32),
    )(pp0, pp1, pp2, pp3)


def kernel(x, edge_index, edge_attr, W_emb, b_emb, W1, b1, W2, b2):
    row = edge_index[0]
    col = edge_index[1]
    ew = edge_attr
    xf = x[:, 0]

    degp = _sc_deg(col, ew).reshape(NW, N)
    dis_row = _reduce32(degp, True)
    dis_flat = dis_row.reshape(N)

    norm, s0p = _sc_norm(row, col, ew, dis_flat)
    s1p = _sc_s1(row, col, norm, xf)

    s0 = _reduce32(s0p.reshape(NW, N), False).reshape(N, 1)
    s1 = _reduce32(s1p.reshape(NW, N), False).reshape(N, 1)

    u = (W_emb[0] @ W1)[None, :]
    v = (b_emb @ W1)[None, :]
    outs = _dense_stage(s0, s1, dis_row.reshape(N, 1), x,
                        u, v, b1[None, :], W2, b2[None, :])
    hw2q, zdq = outs[:4], outs[4:]

    pps = _sc_spmm(row, col, norm, hw2q, zdq)
    return _final_add(*[p.reshape(1, E) for p in pps])[0]


# speedup vs baseline: 17.5979x; 1.0857x over previous
import functools

import jax
import jax.numpy as jnp
from jax import lax
from jax.experimental import pallas as pl
from jax.experimental.pallas import tpu as pltpu
from jax.experimental.pallas import tpu_sc as plsc

N = 50000
E = 800000
NC = 2
NS = 16
NW = NC * NS

EPW = E // NW
EPT = E // NS
CH = 5000
CHP = CH + 16
CH5 = 400

CHD = 400
NPT = N // NS


def _vec_mesh():
    return plsc.VectorSubcoreMesh(core_axis_name="c", subcore_axis_name="s")


_SC_PARAMS = pltpu.CompilerParams(needs_layout_passes=False,
                                  use_tc_tiling_on_sc=False)


def _zero_f32(ref, n):
    def body(i, _):
        ref[pl.ds(i * 16, 16)] = jnp.zeros((16,), jnp.float32)
        return 0
    lax.fori_loop(0, n // 16, body, 0)


def _sc_deg_body(col_hbm, ew_hbm, degp_hbm, colb, ewb, acc, sem):
    cid = lax.axis_index("c")
    sid = lax.axis_index("s")
    wid = sid * NC + cid
    _zero_f32(acc, N)
    base = wid * EPW

    def chunk(ci, _):
        pltpu.async_copy(col_hbm.at[pl.ds(base + ci * CH, CH)], colb.at[pl.ds(0, CH)], sem).wait()
        pltpu.async_copy(ew_hbm.at[pl.ds(base + ci * CH, CH)], ewb.at[pl.ds(0, CH)], sem).wait()

        def body(i, _):
            cvec = colb[pl.ds(i * 16, 16)]
            wvec = ewb[pl.ds(i * 16, 16)]
            m = (i * 16 + lax.iota(jnp.int32, 16)) < CH
            plsc.addupdate_scatter(acc, [cvec], wvec, mask=m)
            return 0
        lax.fori_loop(0, (CH + 15) // 16, body, 0)
        return 0
    lax.fori_loop(0, EPW // CH, chunk, 0)
    pltpu.sync_copy(acc, degp_hbm.at[pl.ds(wid * N, N)])


def _sc_deg(col, ew):
    f = pl.kernel(
        _sc_deg_body,
        out_type=jax.ShapeDtypeStruct((NW * N,), jnp.float32),
        mesh=_vec_mesh(),
        compiler_params=_SC_PARAMS,
        scratch_types=[
            pltpu.VMEM((CHP,), jnp.int32),
            pltpu.VMEM((CHP,), jnp.float32),
            pltpu.VMEM((N,), jnp.float32),
            pltpu.SemaphoreType.DMA,
        ],
    )
    return f(col, ew)


def _sc_norm_body(row_hbm, col_hbm, ew_hbm, dis_hbm, norm_hbm, s0p_hbm,
                  rowb, colb, ewb, normb, disb, acc, sem):
    cid = lax.axis_index("c")
    sid = lax.axis_index("s")
    wid = sid * NC + cid
    pltpu.sync_copy(dis_hbm, disb)
    _zero_f32(acc, N)
    base = wid * EPW

    def chunk(ci, _):
        off = base + ci * CH
        pltpu.async_copy(row_hbm.at[pl.ds(off, CH)], rowb.at[pl.ds(0, CH)], sem).wait()
        pltpu.async_copy(col_hbm.at[pl.ds(off, CH)], colb.at[pl.ds(0, CH)], sem).wait()
        pltpu.async_copy(ew_hbm.at[pl.ds(off, CH)], ewb.at[pl.ds(0, CH)], sem).wait()

        def body(i, _):
            rvec = rowb[pl.ds(i * 16, 16)]
            cvec = colb[pl.ds(i * 16, 16)]
            wvec = ewb[pl.ds(i * 16, 16)]
            m = (i * 16 + lax.iota(jnp.int32, 16)) < CH
            dr = plsc.load_gather(disb, [rvec], mask=m)
            dc = plsc.load_gather(disb, [cvec], mask=m)
            nv = dr * wvec * dc
            normb[pl.ds(i * 16, 16)] = nv
            plsc.addupdate_scatter(acc, [cvec], nv, mask=m)
            return 0
        lax.fori_loop(0, (CH + 15) // 16, body, 0)
        pltpu.sync_copy(normb.at[pl.ds(0, CH)], norm_hbm.at[pl.ds(off, CH)])
        return 0
    lax.fori_loop(0, EPW // CH, chunk, 0)
    pltpu.sync_copy(acc, s0p_hbm.at[pl.ds(wid * N, N)])


def _sc_norm(row, col, ew, dis):
    f = pl.kernel(
        _sc_norm_body,
        out_type=[
            jax.ShapeDtypeStruct((E,), jnp.float32),
            jax.ShapeDtypeStruct((NW * N,), jnp.float32),
        ],
        mesh=_vec_mesh(),
        compiler_params=_SC_PARAMS,
        scratch_types=[
            pltpu.VMEM((CHP,), jnp.int32),
            pltpu.VMEM((CHP,), jnp.int32),
            pltpu.VMEM((CHP,), jnp.float32),
            pltpu.VMEM((CHP,), jnp.float32),
            pltpu.VMEM((N,), jnp.float32),
            pltpu.VMEM((N,), jnp.float32),
            pltpu.SemaphoreType.DMA,
        ],
    )
    return f(row, col, ew, dis)


def _sc_s1_body(row_hbm, col_hbm, norm_hbm, x_hbm, s1p_hbm,
                rowb, colb, normb, xb, acc, sem):
    cid = lax.axis_index("c")
    sid = lax.axis_index("s")
    wid = sid * NC + cid
    pltpu.sync_copy(x_hbm, xb)
    _zero_f32(acc, N)
    base = wid * EPW

    def chunk(ci, _):
        off = base + ci * CH
        pltpu.async_copy(row_hbm.at[pl.ds(off, CH)], rowb.at[pl.ds(0, CH)], sem).wait()
        pltpu.async_copy(col_hbm.at[pl.ds(off, CH)], colb.at[pl.ds(0, CH)], sem).wait()
        pltpu.async_copy(norm_hbm.at[pl.ds(off, CH)], normb.at[pl.ds(0, CH)], sem).wait()

        def body(i, _):
            rvec = rowb[pl.ds(i * 16, 16)]
            cvec = colb[pl.ds(i * 16, 16)]
            nvec = normb[pl.ds(i * 16, 16)]
            m = (i * 16 + lax.iota(jnp.int32, 16)) < CH
            xr = plsc.load_gather(xb, [rvec], mask=m)
            plsc.addupdate_scatter(acc, [cvec], nvec * xr, mask=m)
            return 0
        lax.fori_loop(0, (CH + 15) // 16, body, 0)
        return 0
    lax.fori_loop(0, EPW // CH, chunk, 0)
    pltpu.sync_copy(acc, s1p_hbm.at[pl.ds(wid * N, N)])


def _sc_s1(row, col, norm, x):
    f = pl.kernel(
        _sc_s1_body,
        out_type=jax.ShapeDtypeStruct((NW * N,), jnp.float32),
        mesh=_vec_mesh(),
        compiler_params=_SC_PARAMS,
        scratch_types=[
            pltpu.VMEM((CHP,), jnp.int32),
            pltpu.VMEM((CHP,), jnp.int32),
            pltpu.VMEM((CHP,), jnp.float32),
            pltpu.VMEM((N,), jnp.float32),
            pltpu.VMEM((N,), jnp.float32),
            pltpu.SemaphoreType.DMA,
        ],
    )
    return f(row, col, norm, x)


def _reduce32(parts, rsqrt_of_1p):
    def body(p_ref, o_ref):
        s = jnp.sum(p_ref[...], axis=0, keepdims=True)
        if rsqrt_of_1p:
            o_ref[...] = lax.rsqrt(1.0 + s)
        else:
            o_ref[...] = s
    return pl.pallas_call(
        body,
        grid=(1,),
        in_specs=[pl.BlockSpec((NW, N), lambda i: (0, 0))],
        out_specs=pl.BlockSpec((1, N), lambda i: (0, 0)),
        out_shape=jax.ShapeDtypeStruct((1, N), jnp.float32),
    )(parts)


def _dense_body(s0_ref, s1_ref, dis_ref, x_ref, u_ref, v_ref, b1_ref, w2_ref,
                b2_ref, *out_refs):
    d2 = dis_ref[...] * dis_ref[...]
    s0 = s0_ref[...] + d2
    s1 = s1_ref[...] + d2 * x_ref[...]
    h2 = jnp.maximum(s1 * u_ref[...] + s0 * v_ref[...] + b1_ref[...], 0.0)
    hw2 = jnp.dot(h2, w2_ref[...], preferred_element_type=jnp.float32)
    zd = d2 * hw2 + b2_ref[...]
    for q in range(4):
        out_refs[q][...] = hw2[:, q * 16:(q + 1) * 16]
        out_refs[4 + q][...] = zd[:, q * 16:(q + 1) * 16]


def _dense_stage(s0, s1, dis, x, u, v, b1, W2, b2):
    nb = 2000
    col_spec = pl.BlockSpec((nb, 1), lambda i: (i, 0))
    w_spec = lambda r, c: pl.BlockSpec((r, c), lambda i: (0, 0))
    quarter = pl.BlockSpec((nb, 16), lambda i: (i, 0))
    return pl.pallas_call(
        _dense_body,
        grid=(N // nb,),
        in_specs=[col_spec, col_spec, col_spec, col_spec,
                  w_spec(1, 128), w_spec(1, 128), w_spec(1, 128),
                  w_spec(128, 64), w_spec(1, 64)],
        out_specs=[quarter] * 8,
        out_shape=[jax.ShapeDtypeStruct((N, 16), jnp.float32)] * 8,
    )(s0, s1, dis, x, u, v, b1, W2, b2)


ZP = 400
NZP = N // ZP
NCH = EPT // CH5
NCHD = EPT // CHD


def _sc_spmm_body(row_hbm, col_hbm, norm_hbm,
                  hw2q0, hw2q1, hw2q2, hw2q3, zdq0, zdq1, zdq2, zdq3,
                  pp0, pp1, pp2, pp3,
                  rowb0, colb0, normb0, rowb1, colb1, normb1,
                  rbuf0, rbuf1, dbuf0, dbuf1, pbuf0, pbuf1, zsh,
                  semz, semi0, semi1, semg0, semg1, semo):
    cid = lax.axis_index("c")
    sid = lax.axis_index("s")
    ebase = sid * EPT
    rowb = (rowb0, rowb1)
    colb = (colb0, colb1)
    normb = (normb0, normb1)
    rbuf = (rbuf0, rbuf1)
    pbuf = (pbuf0, pbuf1)
    semi = (semi0, semi1)
    semg = (semg0, semg1)

    def zinit(zd_hbm):
        for pi in range((NZP + NS - 1) // NS):
            p = sid + NS * pi

            @pl.when(p < NZP)
            def _():
                pltpu.async_copy(zd_hbm.at[pl.ds(p * ZP, ZP)],
                                 rbuf0.at[pl.ds(0, ZP)], semz).wait()
                pltpu.sync_copy(rbuf0.at[pl.ds(0, ZP)],
                                zsh.at[pl.ds(p * ZP, ZP)])

    def decode(pp_hbm):
        def d_issue_idx(j, b):
            off = ebase + j * CHD
            pltpu.async_copy(row_hbm.at[pl.ds(off, CHD)], rowb[b], semi[b])
            pltpu.async_copy(col_hbm.at[pl.ds(off, CHD)], colb[b], semi[b])

        def d_wait_idx(b):
            pltpu.make_async_copy(row_hbm.at[pl.ds(0, CHD)], rowb[b],
                                  semi[b]).wait()
            pltpu.make_async_copy(col_hbm.at[pl.ds(0, CHD)], colb[b],
                                  semi[b]).wait()

        gr = (rbuf0, dbuf0)
        gc = (rbuf1, dbuf1)

        def d_issue_gathers(b):
            pltpu.async_copy(zsh.at[rowb[b]], gr[b].at[pl.ds(0, CHD)], semg0)
            pltpu.async_copy(zsh.at[colb[b]], gc[b].at[pl.ds(0, CHD)], semg1)

        def d_wait_gathers(b):
            pltpu.make_async_copy(zsh.at[pl.ds(0, CHD)],
                                  gr[b].at[pl.ds(0, CHD)], semg0).wait()
            pltpu.make_async_copy(zsh.at[pl.ds(0, CHD)],
                                  gc[b].at[pl.ds(0, CHD)], semg1).wait()

        def d_step(j, a):
            b = 1 - a
            d_wait_gathers(a)

            @pl.when(j + 1 < NCHD)
            def _():
                d_wait_idx(b)
                d_issue_gathers(b)

            @pl.when(j + 2 < NCHD)
            def _():
                d_issue_idx(j + 2, a)

            @pl.when(j >= 2)
            def _():
                pltpu.make_async_copy(row_hbm.at[pl.ds(0, CHD)],
                                      pbuf[a], semo).wait()

            pb = pbuf[a]
            bra = gr[a]
            brb = gc[a]

            def grp(g, _):
                jvec = g * 16 + lax.iota(jnp.int32, 16)
                acc = jnp.zeros((16,), jnp.float32)
                for k in range(16):
                    kvec = jnp.full((16,), k, jnp.int32)
                    acc = acc + (plsc.load_gather(bra, [jvec, kvec]) *
                                 plsc.load_gather(brb, [jvec, kvec]))
                pb[pl.ds(g * 16, 16)] = acc
                return 0
            lax.fori_loop(0, CHD // 16, grp, 0)

            pltpu.async_copy(pb, pp_hbm.at[pl.ds(ebase + j * CHD, CHD)], semo)

        d_issue_idx(0, 0)
        d_issue_idx(1, 1)
        d_wait_idx(0)
        d_issue_gathers(0)

        def d_pair(i, _):
            d_step(2 * i, 0)
            d_step(2 * i + 1, 1)
            return 0
        lax.fori_loop(0, NCHD // 2, d_pair, 0)
        d_step(NCHD - 1, (NCHD - 1) % 2)
        for _ in range(2):
            pltpu.make_async_copy(row_hbm.at[pl.ds(0, CHD)], pbuf0,
                                  semo).wait()

    def scale_rows(buf, nrm):
        def grp(g, _):
            nv = nrm[pl.ds(g * 16, 16)]
            for j in range(16):
                e = g * 16 + j
                buf[e, pl.ds(0, 16)] = buf[e, pl.ds(0, 16)] * nv[j]
            return 0
        lax.fori_loop(0, CH5 // 16, grp, 0)

    def spmm(hw2_hbm):
        def issue_idx(j, b):
            off = ebase + j * CH5
            pltpu.async_copy(row_hbm.at[pl.ds(off, CH5)], rowb[b], semi[b])
            pltpu.async_copy(col_hbm.at[pl.ds(off, CH5)], colb[b], semi[b])
            pltpu.async_copy(norm_hbm.at[pl.ds(off, CH5)],
                             normb[b].at[pl.ds(0, CH5)], semi[b])

        def wait_idx(b):
            pltpu.make_async_copy(row_hbm.at[pl.ds(0, CH5)], rowb[b],
                                  semi[b]).wait()
            pltpu.make_async_copy(col_hbm.at[pl.ds(0, CH5)], colb[b],
                                  semi[b]).wait()
            pltpu.make_async_copy(norm_hbm.at[pl.ds(0, CH5)],
                                  normb[b].at[pl.ds(0, CH5)], semi[b]).wait()

        def issue_gather(b):
            pltpu.async_copy(hw2_hbm.at[rowb[b]], rbuf[b].at[pl.ds(0, CH5)],
                             semg[b])

        def wait_gather(b):
            pltpu.make_async_copy(hw2_hbm.at[pl.ds(0, CH5)],
                                  rbuf[b].at[pl.ds(0, CH5)], semg[b]).wait()

        def step(j, a):
            b = 1 - a
            wait_gather(a)

            @pl.when(j + 1 < NCH)
            def _():
                wait_idx(b)
                issue_gather(b)

            scale_rows(rbuf[a], normb[a])
            pltpu.sync_copy(rbuf[a].at[pl.ds(0, CH5)],
                            zsh.at[colb[a]], add=True)

            @pl.when(j + 2 < NCH)
            def _():
                issue_idx(j + 2, a)

        issue_idx(0, 0)
        issue_idx(1, 1)
        wait_idx(0)
        issue_gather(0)

        def pair(i, _):
            step(2 * i, 0)
            step(2 * i + 1, 1)
            return 0
        lax.fori_loop(0, NCH // 2, pair, 0)
        step(NCH - 1, (NCH - 1) % 2)

    def quarter_pass(zd_hbm, hw2_hbm, pp_hbm):
        zinit(zd_hbm)
        plsc.subcore_barrier()
        spmm(hw2_hbm)
        plsc.subcore_barrier()
        decode(pp_hbm)
        plsc.subcore_barrier()

    @pl.when(cid == 0)
    def _():
        quarter_pass(zdq0, hw2q0, pp0)
        quarter_pass(zdq1, hw2q1, pp1)

    @pl.when(cid == 1)
    def _():
        quarter_pass(zdq2, hw2q2, pp2)
        quarter_pass(zdq3, hw2q3, pp3)


def _sc_spmm(row, col, norm, hw2q, zdq):
    f = pl.kernel(
        _sc_spmm_body,
        out_type=[jax.ShapeDtypeStruct((E,), jnp.float32)] * 4,
        mesh=_vec_mesh(),
        compiler_params=_SC_PARAMS,
        scratch_types=[
            pltpu.VMEM((CH5,), jnp.int32),
            pltpu.VMEM((CH5,), jnp.int32),
            pltpu.VMEM((CH5 + 16,), jnp.float32),
            pltpu.VMEM((CH5,), jnp.int32),
            pltpu.VMEM((CH5,), jnp.int32),
            pltpu.VMEM((CH5 + 16,), jnp.float32),
            pltpu.VMEM((CH5 + 16, 16), jnp.float32),
            pltpu.VMEM((CH5 + 16, 16), jnp.float32),
            pltpu.VMEM((CHD, 16), jnp.float32),
            pltpu.VMEM((CHD, 16), jnp.float32),
            pltpu.VMEM((CHD,), jnp.float32),
            pltpu.VMEM((CHD,), jnp.float32),
            pltpu.VMEM_SHARED((N, 16), jnp.float32),
            pltpu.SemaphoreType.DMA,
            pltpu.SemaphoreType.DMA,
            pltpu.SemaphoreType.DMA,
            pltpu.SemaphoreType.DMA,
            pltpu.SemaphoreType.DMA,
            pltpu.SemaphoreType.DMA,
        ],
    )
    return f(row, col, norm, *hw2q, *zdq)


def _final_add(pp0, pp1, pp2, pp3):
    def body(a_ref, b_ref, c_ref, d_ref, o_ref):
        o_ref[...] = ((a_ref[...] + b_ref[...]) +
                      (c_ref[...] + d_ref[...]))
    eb = 80000
    spec = pl.BlockSpec((1, eb), lambda i: (0, i))
    return pl.pallas_call(
        body,
        grid=(E // eb,),
        in_specs=[spec] * 4,
        out_specs=spec,
        out_shape=jax.ShapeDtypeStruct((1, E), jnp.float32),
    )(pp0, pp1, pp2, pp3)


def kernel(x, edge_index, edge_attr, W_emb, b_emb, W1, b1, W2, b2):
    row = edge_index[0]
    col = edge_index[1]
    ew = edge_attr
    xf = x[:, 0]

    degp = _sc_deg(col, ew).reshape(NW, N)
    dis_row = _reduce32(degp, True)
    dis_flat = dis_row.reshape(N)

    norm, s0p = _sc_norm(row, col, ew, dis_flat)
    s1p = _sc_s1(row, col, norm, xf)

    s0 = _reduce32(s0p.reshape(NW, N), False).reshape(N, 1)
    s1 = _reduce32(s1p.reshape(NW, N), False).reshape(N, 1)

    u = (W_emb[0] @ W1)[None, :]
    v = (b_emb @ W1)[None, :]
    outs = _dense_stage(s0, s1, dis_row.reshape(N, 1), x,
                        u, v, b1[None, :], W2, b2[None, :])
    hw2q, zdq = outs[:4], outs[4:]

    pps = _sc_spmm(row, col, norm, hw2q, zdq)
    return _final_add(*[p.reshape(1, E) for p in pps])[0]
